# baseline (speedup 1.0000x reference)
import jax
import jax.numpy as jnp
from jax.experimental import pallas as pl
from jax.experimental.pallas import tpu as pltpu


NUM_ROWS = 100000
EMBED = 64
NUM_CHUNKS = 8
CHUNK = NUM_ROWS // NUM_CHUNKS


def _dma_copy_kernel(w_hbm, o_hbm, sems):
    copies = []
    for i in range(NUM_CHUNKS):
        sl = pl.ds(i * CHUNK, CHUNK)
        copies.append(
            pltpu.make_async_copy(w_hbm.at[sl, :], o_hbm.at[sl, :], sems.at[i])
        )
    for c in copies:
        c.start()
    for c in copies:
        c.wait()


def kernel(x, W):
    del x
    return pl.pallas_call(
        _dma_copy_kernel,
        in_specs=[pl.BlockSpec(memory_space=pltpu.MemorySpace.HBM)],
        out_specs=pl.BlockSpec(memory_space=pltpu.MemorySpace.HBM),
        out_shape=jax.ShapeDtypeStruct((NUM_ROWS, EMBED), jnp.float32),
        scratch_shapes=[pltpu.SemaphoreType.DMA((NUM_CHUNKS,))],
    )(W)


# speedup vs baseline: 15.6909x; 15.6909x over previous
import jax
import jax.numpy as jnp
from jax.experimental import pallas as pl
from jax.experimental.pallas import tpu as pltpu


NUM_ROWS = 100000
EMBED = 64
BLOCK_ROWS = 25000


def _copy_kernel(w_ref, o_ref):
    o_ref[...] = w_ref[...]


def kernel(x, W):
    del x
    return pl.pallas_call(
        _copy_kernel,
        grid=(NUM_ROWS // BLOCK_ROWS,),
        in_specs=[pl.BlockSpec((BLOCK_ROWS, EMBED), lambda i: (i, 0))],
        out_specs=pl.BlockSpec((BLOCK_ROWS, EMBED), lambda i: (i, 0)),
        out_shape=jax.ShapeDtypeStruct((NUM_ROWS, EMBED), jnp.float32),
    )(W)


# speedup vs baseline: 90.3076x; 5.7554x over previous
import jax
import jax.numpy as jnp
from jax.experimental import pallas as pl
from jax.experimental.pallas import tpu as pltpu


NUM_ROWS = 100000
EMBED = 64
BLOCK_SUB = 8


def _copy_kernel(w_ref, o_ref):
    o_ref[...] = w_ref[...]


def kernel(x, W):
    del x
    Wt = W.T
    out_t = pl.pallas_call(
        _copy_kernel,
        grid=(EMBED // BLOCK_SUB,),
        in_specs=[pl.BlockSpec((BLOCK_SUB, NUM_ROWS), lambda i: (i, 0))],
        out_specs=pl.BlockSpec((BLOCK_SUB, NUM_ROWS), lambda i: (i, 0)),
        out_shape=jax.ShapeDtypeStruct((EMBED, NUM_ROWS), jnp.float32),
    )(Wt)
    return out_t.T


# speedup vs baseline: 96.3571x; 1.0670x over previous
import jax
import jax.numpy as jnp
from jax.experimental import pallas as pl
from jax.experimental.pallas import tpu as pltpu


NUM_ROWS = 100000
EMBED = 64
BLOCK_SUB = 16


def _copy_kernel(w_ref, o_ref):
    o_ref[...] = w_ref[...]


def kernel(x, W):
    del x
    Wt = W.T
    out_t = pl.pallas_call(
        _copy_kernel,
        grid=(EMBED // BLOCK_SUB,),
        in_specs=[pl.BlockSpec((BLOCK_SUB, NUM_ROWS), lambda i: (i, 0))],
        out_specs=pl.BlockSpec((BLOCK_SUB, NUM_ROWS), lambda i: (i, 0)),
        out_shape=jax.ShapeDtypeStruct((EMBED, NUM_ROWS), jnp.float32),
    )(Wt)
    return out_t.T


# speedup vs baseline: 101.8590x; 1.0571x over previous
import jax
import jax.numpy as jnp
from jax.experimental import pallas as pl
from jax.experimental.pallas import tpu as pltpu


NUM_ROWS = 100000
EMBED = 64
BLOCK_SUB = 32


def _copy_kernel(w_ref, o_ref):
    o_ref[...] = w_ref[...]


def kernel(x, W):
    del x
    Wt = W.T
    out_t = pl.pallas_call(
        _copy_kernel,
        grid=(EMBED // BLOCK_SUB,),
        in_specs=[pl.BlockSpec((BLOCK_SUB, NUM_ROWS), lambda i: (i, 0))],
        out_specs=pl.BlockSpec((BLOCK_SUB, NUM_ROWS), lambda i: (i, 0)),
        out_shape=jax.ShapeDtypeStruct((EMBED, NUM_ROWS), jnp.float32),
    )(Wt)
    return out_t.T


# speedup vs baseline: 101.9505x; 1.0009x over previous
import jax
import jax.numpy as jnp
from jax.experimental import pallas as pl
from jax.experimental.pallas import tpu as pltpu


NUM_ROWS = 100000
EMBED = 64
BLOCK_SUB = 32


def _copy_kernel(w_ref, o_ref):
    o_ref[...] = w_ref[...]


def kernel(x, W):
    del x
    Wt = W.T
    out_t = pl.pallas_call(
        _copy_kernel,
        grid=(EMBED // BLOCK_SUB,),
        in_specs=[pl.BlockSpec((BLOCK_SUB, NUM_ROWS), lambda i: (i, 0))],
        out_specs=pl.BlockSpec((BLOCK_SUB, NUM_ROWS), lambda i: (i, 0)),
        out_shape=jax.ShapeDtypeStruct((EMBED, NUM_ROWS), jnp.float32),
        compiler_params=pltpu.CompilerParams(
            dimension_semantics=("parallel",),
        ),
    )(Wt)
    return out_t.T
